# Initial kernel scaffold; baseline (speedup 1.0000x reference)
import jax
import jax.numpy as jnp
from jax.experimental import pallas as pl


def kernel(x, edge_index, batch, target, w1, b1, w2, b2, w3, b3, g1w, g1b, g2w, g2b, emb, cw, cb, xw, xb, f1w, f1b, f2w, f2b, ow, ob):
    raise NotImplementedError("write your pallas kernel here")


# speedup vs baseline: 2.2769x; 2.2769x over previous
import functools

import jax
import jax.numpy as jnp
from jax import lax
from jax.experimental import pallas as pl
from jax.experimental.pallas import tpu as pltpu

N = 50000
E = 800000
B = 256


def _gcn_conv(x, edge_index, W, b, dinv):
    src = edge_index[0]
    dst = edge_index[1]
    h = x @ W
    hp = h * dinv[:, None]
    msg = hp[src]
    s = jnp.zeros_like(h).at[dst].add(msg)
    agg = dinv[:, None] * (s + hp)
    return agg + b


def _head_body(g_ref, xt_ref, f1w_ref, f1b_ref, f2w_ref, f2b_ref, ow_ref,
               ob_ref, o_ref):
    g = g_ref[...]
    xt = xt_ref[...]
    y = jnp.dot(g, f1w_ref[:128, :], preferred_element_type=jnp.float32)
    y = y + jnp.dot(xt, f1w_ref[128:, :], preferred_element_type=jnp.float32)
    y = jax.nn.relu(y + f1b_ref[...])
    z = jax.nn.relu(
        jnp.dot(y, f2w_ref[...], preferred_element_type=jnp.float32)
        + f2b_ref[...])
    o = jnp.dot(z, ow_ref[...], preferred_element_type=jnp.float32)
    o_ref[...] = o + ob_ref[...]


def _head(g, xt, f1w, f1b, f2w, f2b, ow, ob):
    return pl.pallas_call(
        _head_body,
        out_shape=jax.ShapeDtypeStruct((B, 1), jnp.float32),
    )(g, xt, f1w, f1b[None, :], f2w, f2b[None, :], ow, ob[None, :])


def kernel(x, edge_index, batch, target, w1, b1, w2, b2, w3, b3, g1w, g1b,
           g2w, g2b, emb, cw, cb, xw, xb, f1w, f1b, f2w, f2b, ow, ob):
    dst = edge_index[1]
    deg = jnp.ones((N,), jnp.float32).at[dst].add(1.0)
    dinv = lax.rsqrt(deg)

    h = jax.nn.relu(_gcn_conv(x, edge_index, w1, b1, dinv))
    h = jax.nn.relu(_gcn_conv(h, edge_index, w2, b2, dinv))
    h = jax.nn.relu(_gcn_conv(h, edge_index, w3, b3, dinv))
    g = jax.ops.segment_max(h, batch, num_segments=B)
    g = jnp.where(jnp.isfinite(g), g, 0.0)
    g = jax.nn.relu(g @ g1w + g1b)
    g = g @ g2w + g2b

    e = emb[target]
    c = lax.conv_general_dilated(
        e, cw, window_strides=(1,), padding='VALID',
        dimension_numbers=('NCH', 'OIH', 'NCH')) + cb[None, :, None]
    xt = c.reshape(-1, 32 * 121)
    xt = xt @ xw + xb
    return _head(g, xt, f1w, f1b, f2w, f2b, ow, ob)


# speedup vs baseline: 2.4382x; 1.0708x over previous
import functools

import jax
import jax.numpy as jnp
from jax import lax
from jax.experimental import pallas as pl
from jax.experimental.pallas import tpu as pltpu
from jax.experimental.pallas import tpu_sc as plsc

N = 50000
NPAD = 50176
E = 800000
EPAD = 819200
B = 256
ROWS2D = EPAD // 128
TPW = 200
TPT = NPAD // 16


def _make_edge_scatter(nc):
    mesh = plsc.VectorSubcoreMesh(core_axis_name="c", subcore_axis_name="s")

    @functools.partial(
        pl.kernel,
        mesh=mesh,
        out_type=jax.ShapeDtypeStruct((2, nc, NPAD, 32), jnp.float32),
        scratch_types=[
            pltpu.VMEM((TPW, 128), jnp.int32),
            pltpu.VMEM((TPW, 128), jnp.int32),
            pltpu.VMEM((128, 32), jnp.float32),
            pltpu.VMEM((392, 32), jnp.float32),
            pltpu.VMEM_SHARED((NPAD, 32), jnp.float32),
        ],
    )
    def k(hp, src2d, dst2d, out, srcbuf, dstbuf, rows, zbuf, acc):
        c = lax.axis_index("c")
        s = lax.axis_index("s")
        base = c * (ROWS2D // 2) + s * TPW
        pltpu.sync_copy(src2d.at[pl.ds(base, TPW)], srcbuf)
        pltpu.sync_copy(dst2d.at[pl.ds(base, TPW)], dstbuf)

        def zb(i, carry):
            zbuf[i, pl.ds(0, 16)] = jnp.zeros((16,), jnp.float32)
            zbuf[i, pl.ds(16, 16)] = jnp.zeros((16,), jnp.float32)
            return carry

        lax.fori_loop(0, 392, zb, 0)
        slice_lo = s * TPT
        for chunk in range(nc):
            for kk in range(8):
                pltpu.sync_copy(zbuf, acc.at[pl.ds(slice_lo + kk * 392, 392)])
            plsc.subcore_barrier()

            def win(w, carry):
                pltpu.sync_copy(hp.at[chunk].at[srcbuf.at[w]], rows)
                pltpu.sync_copy(rows, acc.at[dstbuf.at[w]], add=True)
                return carry

            lax.fori_loop(0, TPW, win, 0)
            plsc.subcore_barrier()
            pltpu.sync_copy(acc.at[pl.ds(slice_lo, TPT)],
                            out.at[c, chunk, pl.ds(slice_lo, TPT)])

    return k


_edge_scatter = {nc: _make_edge_scatter(nc) for nc in (3, 5, 10)}


def _mm_chunked(a_ch, w4, dinv, nc_in, nc_out):
    def body(a_ref, w_ref, d_ref, o_ref):
        acc = jnp.zeros((512, 32), jnp.float32)
        for cc in range(nc_in):
            acc += jnp.dot(a_ref[cc], w_ref[0, cc],
                           preferred_element_type=jnp.float32)
        o_ref[0] = d_ref[...] * acc

    return pl.pallas_call(
        body,
        grid=(98, nc_out),
        in_specs=[
            pl.BlockSpec((nc_in, 512, 32), lambda i, c: (0, i, 0)),
            pl.BlockSpec((1, nc_in, 32, 32), lambda i, c: (c, 0, 0, 0)),
            pl.BlockSpec((512, 1), lambda i, c: (i, 0)),
        ],
        out_specs=pl.BlockSpec((1, 512, 32), lambda i, c: (c, i, 0)),
        out_shape=jax.ShapeDtypeStruct((nc_out, NPAD, 32), jnp.float32),
    )(a_ch, w4, dinv)


def _act(spart, hp, dinv, b_ch, nc):
    def body(s_ref, h_ref, d_ref, b_ref, o_ref):
        t = s_ref[0, 0] + s_ref[1, 0] + h_ref[0]
        o_ref[0] = jax.nn.relu(d_ref[...] * t + b_ref[0])

    return pl.pallas_call(
        body,
        grid=(98, nc),
        in_specs=[
            pl.BlockSpec((2, 1, 512, 32), lambda i, c: (0, c, i, 0)),
            pl.BlockSpec((1, 512, 32), lambda i, c: (c, i, 0)),
            pl.BlockSpec((512, 1), lambda i, c: (i, 0)),
            pl.BlockSpec((1, 1, 32), lambda i, c: (c, 0, 0)),
        ],
        out_specs=pl.BlockSpec((1, 512, 32), lambda i, c: (c, i, 0)),
        out_shape=jax.ShapeDtypeStruct((nc, NPAD, 32), jnp.float32),
    )(spart, hp, dinv, b_ch)


def _head_body(g_ref, xt_ref, f1w_ref, f1b_ref, f2w_ref, f2b_ref, ow_ref,
               ob_ref, o_ref):
    g = g_ref[...]
    xt = xt_ref[...]
    y = jnp.dot(g, f1w_ref[:128, :], preferred_element_type=jnp.float32)
    y = y + jnp.dot(xt, f1w_ref[128:, :], preferred_element_type=jnp.float32)
    y = jax.nn.relu(y + f1b_ref[...])
    z = jax.nn.relu(
        jnp.dot(y, f2w_ref[...], preferred_element_type=jnp.float32)
        + f2b_ref[...])
    o = jnp.dot(z, ow_ref[...], preferred_element_type=jnp.float32)
    o_ref[...] = o + ob_ref[...]


def _head(g, xt, f1w, f1b, f2w, f2b, ow, ob):
    return pl.pallas_call(
        _head_body,
        out_shape=jax.ShapeDtypeStruct((B, 1), jnp.float32),
    )(g, xt, f1w, f1b[None, :], f2w, f2b[None, :], ow, ob[None, :])


def _pad2(w, r, c):
    return jnp.zeros((r, c), jnp.float32).at[:w.shape[0], :w.shape[1]].set(w)


def kernel(x, edge_index, batch, target, w1, b1, w2, b2, w3, b3, g1w, g1b,
           g2w, g2b, emb, cw, cb, xw, xb, f1w, f1b, f2w, f2b, ow, ob):
    src0 = edge_index[0]
    dst0 = edge_index[1]
    key = jnp.sort((dst0.astype(jnp.uint32) << 16) | src0.astype(jnp.uint32))
    src = (key & jnp.uint32(0xFFFF)).astype(jnp.int32)
    dst = (key >> 16).astype(jnp.int32)
    deg = jnp.ones((NPAD,), jnp.float32).at[dst].add(
        1.0, indices_are_sorted=True)
    dinv = lax.rsqrt(deg)[:, None]

    pad_n = EPAD - E
    src_p = jnp.concatenate([src, jnp.full((pad_n,), N, jnp.int32)])
    dst_p = jnp.concatenate(
        [dst, N + (jnp.arange(pad_n, dtype=jnp.int32) % 128)])
    src2d = src_p.reshape(ROWS2D, 128)
    dst2d = dst_p.reshape(ROWS2D, 128)

    dinv_n = dinv[:N]

    def layer(a, w, b):
        hp = dinv_n * (a @ w)
        s = jnp.zeros_like(hp).at[dst].add(
            hp[src], indices_are_sorted=True)
        return jax.nn.relu(dinv_n * (s + hp) + b)

    a1 = layer(x, w1, b1)
    a2 = layer(a1, w2, b2)
    h = layer(a2, w3, b3)
    g = jax.ops.segment_max(h, batch, num_segments=B)
    g = jnp.where(jnp.isfinite(g), g, 0.0)
    g = jax.nn.relu(g @ g1w + g1b)
    g = g @ g2w + g2b

    e = emb[target]
    c = lax.conv_general_dilated(
        e, cw, window_strides=(1,), padding='VALID',
        dimension_numbers=('NCH', 'OIH', 'NCH')) + cb[None, :, None]
    xt = c.reshape(-1, 32 * 121)
    xt = xt @ xw + xb
    return _head(g, xt, f1w, f1b, f2w, f2b, ow, ob)


# speedup vs baseline: 4.5779x; 1.8776x over previous
import functools

import jax
import jax.numpy as jnp
from jax import lax
from jax.experimental import pallas as pl
from jax.experimental.pallas import tpu as pltpu
from jax.experimental.pallas import tpu_sc as plsc

N = 50000
NPAD = 50176
E = 800000
B = 256
EPAD = E + 288
NW = 32
SENT = 0x3FFFFF


def _seg_gcn(nc):
    mesh = plsc.VectorSubcoreMesh(core_axis_name="c", subcore_axis_name="s")
    scratch = ([pltpu.VMEM((160,), jnp.int32),
                pltpu.VMEM((128,), jnp.int32),
                pltpu.VMEM((128,), jnp.int32),
                pltpu.VMEM((128,), jnp.int32)]
               + [pltpu.VMEM((128, 128), jnp.float32) for _ in range(2 * nc)]
               + [pltpu.VMEM_SHARED((128, 128), jnp.float32)])

    @functools.partial(
        pl.kernel, mesh=mesh,
        out_type=jax.ShapeDtypeStruct((nc, NPAD, 128), jnp.float32),
        scratch_types=scratch,
        compiler_params=pltpu.CompilerParams(needs_layout_passes=False))
    def k(hp, srcs, dsts, esplit, nsplit, out, dstwin, srcwin, esb, nsb,
          *bufs):
        rowb = bufs[:nc]
        outb = bufs[nc:2 * nc]
        zbuf = bufs[2 * nc]
        c = lax.axis_index("c")
        s = lax.axis_index("s")
        w = s * 2 + c
        pltpu.sync_copy(esplit, esb)
        pltpu.sync_copy(nsplit, nsb)
        iota = lax.iota(jnp.int32, 16)
        zf = jnp.zeros((16,), jnp.float32)

        def splat(v):
            return jnp.full((16,), v, jnp.int32)

        def sread(buf, i):
            return buf[pl.ds(i, 16)][0]

        lo = sread(esb, w)
        hi = sread(esb, w + 1)
        nlo = sread(nsb, w)
        nhi = sread(nsb, w + 1)

        def zero_row(i, carry):
            for kk in range(8):
                outb[0][i, pl.ds(kk * 16, 16)] = zf
            return carry

        lax.fori_loop(0, 128, zero_row, 0)

        @pl.when(s == 0)
        def _seed_zbuf():
            pltpu.sync_copy(outb[0], zbuf)

        plsc.subcore_barrier()
        for cc in range(1, nc):
            pltpu.sync_copy(zbuf, outb[cc])

        def zpass(i, carry):
            v = pl.multiple_of(nlo + i * 128, 128)
            for cc in range(nc):
                pltpu.sync_copy(zbuf, out.at[cc, pl.ds(v, 128)])
            return carry

        lax.fori_loop(0, (nhi - nlo) // 128, zpass, 0)

        lo8 = lo & jnp.int32(~7)
        nwin = (hi - lo8 + 127) // 128
        zaccs = tuple(zf for _ in range(nc * 8))

        def window(wi, carry):
            win0, accs0 = carry
            base = lo8 + wi * 128
            base_a = pl.multiple_of(base, 8)
            pltpu.sync_copy(dsts.at[pl.ds(base_a, 160)], dstwin)
            pltpu.sync_copy(srcs.at[pl.ds(base_a, 128)], srcwin)
            for cc in range(nc):
                pltpu.sync_copy(hp.at[cc].at[srcwin], rowb[cc])
            start = jnp.maximum(lo - base, 0)
            limit = jnp.minimum(hi - base, 128)

            def edge(j, ecarry):
                ewin, eaccs = ecarry
                dpair = dstwin[pl.ds(j, 16)]
                d0 = dpair[0]
                d1 = dpair[1]
                acc2 = tuple(
                    eaccs[cc * 8 + kk] + rowb[cc][j, pl.ds(kk * 16, 16)]
                    for cc in range(nc) for kk in range(8))
                flush = d0 != d1

                def adv(v):
                    va = pl.multiple_of(v, 128)
                    for cc in range(nc):
                        pltpu.sync_copy(outb[cc], out.at[cc, pl.ds(va, 128)])
                        pltpu.sync_copy(zbuf, outb[cc])
                    return (d0 // 128) * 128

                ewin = lax.cond(
                    jnp.logical_and(flush, d0 - ewin >= 128),
                    adv, lambda v: v, ewin)

                def store(_):
                    row = splat(d0 & 127)
                    for cc in range(nc):
                        for kk in range(8):
                            plsc.store_scatter(
                                outb[cc], [row, splat(kk * 16) + iota],
                                acc2[cc * 8 + kk])
                    return 0

                lax.cond(flush, store, lambda _: 0, 0)
                acc3 = tuple(jnp.where(flush, zf, a) for a in acc2)
                return ewin, acc3

            return lax.fori_loop(start, limit, edge, (win0, accs0))

        win, _ = lax.fori_loop(0, nwin, window, (nlo, zaccs))

        def final(v):
            va = pl.multiple_of(v, 128)
            for cc in range(nc):
                pltpu.sync_copy(outb[cc], out.at[cc, pl.ds(va, 128)])
            return 0

        lax.cond(win < nhi, final, lambda v: 0, win)

    return k


_seg = {nc: _seg_gcn(nc) for nc in (1, 2, 3)}


def _mm_chunked(a_ch, w4, dinv, nci, nco):
    def body(a_ref, w_ref, d_ref, o_ref):
        acc = jnp.zeros((512, 128), jnp.float32)
        for cc in range(nci):
            acc += jnp.dot(a_ref[cc], w_ref[0, cc],
                           preferred_element_type=jnp.float32)
        o_ref[0] = d_ref[...] * acc

    return pl.pallas_call(
        body,
        grid=(98, nco),
        in_specs=[
            pl.BlockSpec((nci, 512, 128), lambda i, c: (0, i, 0)),
            pl.BlockSpec((1, nci, 128, 128), lambda i, c: (c, 0, 0, 0)),
            pl.BlockSpec((512, 1), lambda i, c: (i, 0)),
        ],
        out_specs=pl.BlockSpec((1, 512, 128), lambda i, c: (c, i, 0)),
        out_shape=jax.ShapeDtypeStruct((nco, NPAD, 128), jnp.float32),
    )(a_ch, w4, dinv)


def _act(sagg, hp, dinv, b_ch, nc):
    def body(s_ref, h_ref, d_ref, b_ref, o_ref):
        o_ref[0] = jax.nn.relu(
            d_ref[...] * (s_ref[0] + h_ref[0]) + b_ref[0])

    return pl.pallas_call(
        body,
        grid=(98, nc),
        in_specs=[
            pl.BlockSpec((1, 512, 128), lambda i, c: (c, i, 0)),
            pl.BlockSpec((1, 512, 128), lambda i, c: (c, i, 0)),
            pl.BlockSpec((512, 1), lambda i, c: (i, 0)),
            pl.BlockSpec((1, 1, 128), lambda i, c: (c, 0, 0)),
        ],
        out_specs=pl.BlockSpec((1, 512, 128), lambda i, c: (c, i, 0)),
        out_shape=jax.ShapeDtypeStruct((nc, NPAD, 128), jnp.float32),
    )(sagg, hp, dinv, b_ch)


def _u_kernel(emb_sh, xwT3):
    def body(e_ref, x_ref, o_ref):
        for o in range(8):
            o_ref[:, o, :] = jnp.dot(e_ref[0], x_ref[:, o, :],
                                     preferred_element_type=jnp.float32)

    return pl.pallas_call(
        body, grid=(8, 4),
        in_specs=[pl.BlockSpec((1, 26, 128), lambda k, o: (k, 0, 0)),
                  pl.BlockSpec((128, 8, 128), lambda k, o: (0, o, 0))],
        out_specs=pl.BlockSpec((26, 8, 128), lambda k, o: (0, k * 4 + o, 0)),
        out_shape=jax.ShapeDtypeStruct((26, 256, 128), jnp.float32),
    )(emb_sh, xwT3)


def _prot_kernel(tgt, cwT, uP, xb2, cbterm):
    def body(t_ref, c_ref, u_ref, xb_ref, cb_ref, o_ref):
        cidx = pl.program_id(0)
        m = (t_ref[...] == cidx).astype(jnp.float32)
        d = jnp.dot(m, c_ref[...], preferred_element_type=jnp.float32)
        p = jnp.dot(d, u_ref[0], preferred_element_type=jnp.float32)

        @pl.when(cidx == 0)
        def _():
            o_ref[...] = p + xb_ref[...] + cb_ref[...]

        @pl.when(cidx != 0)
        def _():
            o_ref[...] = o_ref[...] + p

    return pl.pallas_call(
        body, grid=(26,),
        in_specs=[pl.BlockSpec((256, 1000), lambda c: (0, 0)),
                  pl.BlockSpec((1000, 256), lambda c: (0, 0)),
                  pl.BlockSpec((1, 256, 128), lambda c: (c, 0, 0)),
                  pl.BlockSpec((1, 128), lambda c: (0, 0)),
                  pl.BlockSpec((1, 128), lambda c: (0, 0))],
        out_specs=pl.BlockSpec((256, 128), lambda c: (0, 0)),
        out_shape=jax.ShapeDtypeStruct((256, 128), jnp.float32),
    )(tgt, cwT, uP, xb2, cbterm)


def _head_body(g_ref, xt_ref, g1w_ref, g1b_ref, g2w_ref, g2b_ref, f1w_ref,
               f1b_ref, f2w_ref, f2b_ref, ow_ref, ob_ref, o_ref):
    g = g_ref[...]
    g = jnp.where(jnp.isfinite(g), g, 0.0)
    gg = jax.nn.relu(
        jnp.dot(g, g1w_ref[...], preferred_element_type=jnp.float32)
        + g1b_ref[...])
    gg = (jnp.dot(gg, g2w_ref[...], preferred_element_type=jnp.float32)
          + g2b_ref[...])
    xt = xt_ref[...]
    y = jnp.dot(gg, f1w_ref[:128, :], preferred_element_type=jnp.float32)
    y = y + jnp.dot(xt, f1w_ref[128:, :], preferred_element_type=jnp.float32)
    y = jax.nn.relu(y + f1b_ref[...])
    z = jax.nn.relu(
        jnp.dot(y, f2w_ref[...], preferred_element_type=jnp.float32)
        + f2b_ref[...])
    o = jnp.dot(z, ow_ref[...], preferred_element_type=jnp.float32)
    o_ref[...] = o + ob_ref[...]


def _head(g, xt, g1w, g1b, g2w, g2b, f1w, f1b, f2w, f2b, ow, ob):
    return pl.pallas_call(
        _head_body,
        out_shape=jax.ShapeDtypeStruct((B, 1), jnp.float32),
    )(g, xt, g1w, g1b[None, :], g2w, g2b[None, :], f1w, f1b[None, :],
      f2w, f2b[None, :], ow, ob[None, :])


def _pad2(w, r, c):
    return jnp.zeros((r, c), jnp.float32).at[:w.shape[0], :w.shape[1]].set(w)


def _wch(w, rin, cout, nci, nco):
    return (_pad2(w, rin, cout).reshape(nci, 128, nco, 128)
            .transpose(2, 0, 1, 3))


def kernel(x, edge_index, batch, target, w1, b1, w2, b2, w3, b3, g1w, g1b,
           g2w, g2b, emb, cw, cb, xw, xb, f1w, f1b, f2w, f2b, ow, ob):
    src0 = edge_index[0]
    dst0 = edge_index[1]
    key = jnp.sort((dst0.astype(jnp.uint32) << 16) | src0.astype(jnp.uint32))
    src = (key & jnp.uint32(0xFFFF)).astype(jnp.int32)
    dst = (key >> 16).astype(jnp.int32)
    srcs = jnp.concatenate([src, jnp.zeros((EPAD - E,), jnp.int32)])
    dsts = jnp.concatenate([dst, jnp.full((EPAD - E,), SENT, jnp.int32)])

    deg = jnp.ones((NPAD,), jnp.float32).at[dst].add(
        1.0, indices_are_sorted=True)
    dinv = lax.rsqrt(deg)[:, None]

    cb_ = jnp.searchsorted(dst, jnp.arange(393, dtype=jnp.int32) * 128)
    bw = jnp.searchsorted(cb_, (jnp.arange(33, dtype=jnp.int32) * E) // NW)
    bw = jnp.clip(bw, 0, 392).at[0].set(0).at[NW].set(392)
    nsplit = jnp.zeros((128,), jnp.int32).at[:33].set(
        (bw * 128).astype(jnp.int32))
    esplit = jnp.zeros((128,), jnp.int32).at[:33].set(
        cb_[bw].astype(jnp.int32))

    x_ch = jnp.zeros((1, NPAD, 128), jnp.float32).at[0, :N, :78].set(x)
    w1c = _wch(w1, 128, 128, 1, 1)
    w2c = _wch(w2, 128, 256, 1, 2)
    w3c = _wch(w3, 256, 384, 2, 3)
    b1c = _pad2(b1[None, :], 1, 128).reshape(1, 1, 128)
    b2c = _pad2(b2[None, :], 1, 256).reshape(2, 1, 128)
    b3c = _pad2(b3[None, :], 1, 384).reshape(3, 1, 128)

    hp1 = _mm_chunked(x_ch, w1c, dinv, 1, 1)
    a1 = _act(_seg[1](hp1, srcs, dsts, esplit, nsplit), hp1, dinv, b1c, 1)
    hp2 = _mm_chunked(a1, w2c, dinv, 1, 2)
    a2 = _act(_seg[2](hp2, srcs, dsts, esplit, nsplit), hp2, dinv, b2c, 2)
    hp3 = _mm_chunked(a2, w3c, dinv, 2, 3)
    a3 = _act(_seg[3](hp3, srcs, dsts, esplit, nsplit), hp3, dinv, b3c, 3)

    h = a3.transpose(1, 0, 2).reshape(NPAD, 384)[:N, :312]
    g = jax.ops.segment_max(h, batch, num_segments=B)

    xw2 = xw.reshape(32, 121, 128)
    xwT3 = jnp.zeros((128, 32, 128), jnp.float32).at[:121].set(
        xw2.transpose(1, 0, 2))
    emb_sh = jnp.stack(
        [jnp.pad(emb[:, k:], ((0, 0), (0, k))) for k in range(8)])
    cwT = cw.transpose(1, 2, 0).reshape(1000, 256)
    cbterm = cb[None, :] @ xw2.sum(1)
    uP = _u_kernel(emb_sh, xwT3)
    xt = _prot_kernel(target, cwT, uP, xb[None, :], cbterm)
    return _head(g, xt, g1w, g1b, g2w, g2b, f1w, f1b, f2w, f2b, ow, ob)
